# baseline (speedup 1.0000x reference)
import functools

import jax
import jax.numpy as jnp
from jax import lax
from jax.experimental import pallas as pl
from jax.experimental.pallas import tpu as pltpu
from jax.experimental.pallas import tpu_sc as plsc

EPS = 1e-5
_NC = 2
_NS = 16
_FC = 128


def _leaky(v):
    return jnp.where(v >= 0, v, 0.2 * v)


def _ln(h, g, b):
    mu = jnp.mean(h, axis=-1, keepdims=True)
    d = h - mu
    var = jnp.mean(d * d, axis=-1, keepdims=True)
    return d * lax.rsqrt(var + EPS) * g + b


def _row_tile(n, cap=1024):
    for t in range(cap, 7, -8):
        if n % t == 0:
            return t
    return n


def _ab_body(x_ref, w_ref, a_ref, b_ref):
    h = jnp.dot(x_ref[...], w_ref[...], preferred_element_type=jnp.float32)
    d = h.shape[1] // 2
    a_ref[...] = h[:, :d].astype(jnp.bfloat16)
    b_ref[...] = h[:, d:].astype(jnp.bfloat16)


def _ab_call(x, wcat):
    n, d_in = x.shape
    d2 = wcat.shape[1]
    d = d2 // 2
    tn = _row_tile(n)
    return pl.pallas_call(
        _ab_body,
        grid=(n // tn,),
        in_specs=[
            pl.BlockSpec((tn, d_in), lambda i: (i, 0)),
            pl.BlockSpec((d_in, d2), lambda i: (0, 0)),
        ],
        out_specs=[
            pl.BlockSpec((tn, d), lambda i: (i, 0)),
            pl.BlockSpec((tn, d), lambda i: (i, 0)),
        ],
        out_shape=[
            jax.ShapeDtypeStruct((n, d), jnp.bfloat16),
            jax.ShapeDtypeStruct((n, d), jnp.bfloat16),
        ],
    )(x, wcat)


def _edge_tail_body(ga_ref, gb_ref, b1_ref, w2_ref, b2_ref, w3_ref, g_ref,
                    be_ref, out_ref):
    h1 = _leaky(ga_ref[...].astype(jnp.float32)
                + gb_ref[...].astype(jnp.float32) + b1_ref[...])
    h2 = _leaky(jnp.dot(h1, w2_ref[...], preferred_element_type=jnp.float32)
                + b2_ref[...])
    h3 = jnp.dot(h2, w3_ref[...], preferred_element_type=jnp.float32)
    msg = _ln(h3, g_ref[...], be_ref[...])
    for c4 in range(out_ref.shape[0]):
        out_ref[c4] = msg[:, c4 * _FC:(c4 + 1) * _FC]


def _edge_tail_call(ga, gb, p):
    e, d = ga.shape
    np4 = d // _FC
    te = _row_tile(e)
    vec = lambda v: v.reshape(1, -1)
    return pl.pallas_call(
        _edge_tail_body,
        grid=(e // te,),
        in_specs=[
            pl.BlockSpec((te, d), lambda i: (i, 0)),
            pl.BlockSpec((te, d), lambda i: (i, 0)),
            pl.BlockSpec((1, d), lambda i: (0, 0)),
            pl.BlockSpec((d, d), lambda i: (0, 0)),
            pl.BlockSpec((1, d), lambda i: (0, 0)),
            pl.BlockSpec((d, d), lambda i: (0, 0)),
            pl.BlockSpec((1, d), lambda i: (0, 0)),
            pl.BlockSpec((1, d), lambda i: (0, 0)),
        ],
        out_specs=pl.BlockSpec((np4, te, _FC), lambda i: (0, i, 0)),
        out_shape=jax.ShapeDtypeStruct((np4, e, _FC), jnp.float32),
    )(ga, gb, vec(p['b1']), p['W2'], vec(p['b2']), p['W3'], vec(p['g']),
      vec(p['be']))


def _gather_add_call(a, b, dest_g, src_g):
    n, d = a.shape
    nw = _NC * _NS
    _, nb, bs = dest_g.shape
    epw = nb * bs
    e = epw * nw

    mesh = plsc.VectorSubcoreMesh(core_axis_name="c", subcore_axis_name="s")

    @functools.partial(
        pl.kernel,
        out_type=[
            jax.ShapeDtypeStruct((e, d), jnp.int32),
            jax.ShapeDtypeStruct((e, d), jnp.int32),
        ],
        mesh=mesh,
        scratch_types=[
            pltpu.VMEM((nb, bs), jnp.int32),
            pltpu.VMEM((nb, bs), jnp.int32),
            pltpu.VMEM((bs, d), jnp.int32),
            pltpu.VMEM((bs, d), jnp.int32),
            pltpu.VMEM((bs, d), jnp.int32),
            pltpu.VMEM((bs, d), jnp.int32),
        ] + [pltpu.SemaphoreType.DMA] * 8,
    )
    def gath(a_hbm, b_hbm, di_hbm, si_hbm, outa_hbm, outb_hbm, di_v, si_v,
             a0, a1, b0, b1, semIA0, semIA1, semIB0, semIB1, semOA0, semOA1,
             semOB0, semOB1):
        c = lax.axis_index("c")
        s = lax.axis_index("s")
        wid = s * _NC + c
        base = wid * epw
        pltpu.sync_copy(di_hbm.at[wid], di_v)
        pltpu.sync_copy(si_hbm.at[wid], si_v)

        j0 = nb % 2
        if j0:
            pltpu.sync_copy(a_hbm.at[di_v.at[0]], a0)
            pltpu.sync_copy(a0, outa_hbm.at[pl.ds(base, bs)])
            pltpu.sync_copy(b_hbm.at[si_v.at[0]], b0)
            pltpu.sync_copy(b0, outb_hbm.at[pl.ds(base, bs)])

        pltpu.async_copy(a_hbm.at[di_v.at[j0]], a0, semIA0)
        pltpu.async_copy(b_hbm.at[si_v.at[j0]], b0, semIB0)
        pltpu.async_copy(a_hbm.at[di_v.at[j0 + 1]], a1, semIA1)
        pltpu.async_copy(b_hbm.at[si_v.at[j0 + 1]], b1, semIB1)

        def body(i, _):
            j = j0 + 2 * i
            pltpu.make_async_copy(a_hbm.at[di_v.at[0]], a0, semIA0).wait()
            pltpu.async_copy(a0, outa_hbm.at[pl.ds(base + j * bs, bs)],
                             semOA0)
            pltpu.make_async_copy(b_hbm.at[si_v.at[0]], b0, semIB0).wait()
            pltpu.async_copy(b0, outb_hbm.at[pl.ds(base + j * bs, bs)],
                             semOB0)
            pltpu.make_async_copy(a_hbm.at[di_v.at[0]], a1, semIA1).wait()
            pltpu.async_copy(a1, outa_hbm.at[pl.ds(base + (j + 1) * bs, bs)],
                             semOA1)
            pltpu.make_async_copy(b_hbm.at[si_v.at[0]], b1, semIB1).wait()
            pltpu.async_copy(b1, outb_hbm.at[pl.ds(base + (j + 1) * bs, bs)],
                             semOB1)

            @pl.when(j + 2 < nb)
            def _():
                pltpu.make_async_copy(a0, outa_hbm.at[pl.ds(base, bs)],
                                      semOA0).wait()
                pltpu.async_copy(a_hbm.at[di_v.at[j + 2]], a0, semIA0)
                pltpu.make_async_copy(b0, outb_hbm.at[pl.ds(base, bs)],
                                      semOB0).wait()
                pltpu.async_copy(b_hbm.at[si_v.at[j + 2]], b0, semIB0)

            @pl.when(j + 3 < nb)
            def _():
                pltpu.make_async_copy(a1, outa_hbm.at[pl.ds(base, bs)],
                                      semOA1).wait()
                pltpu.async_copy(a_hbm.at[di_v.at[j + 3]], a1, semIA1)
                pltpu.make_async_copy(b1, outb_hbm.at[pl.ds(base, bs)],
                                      semOB1).wait()
                pltpu.async_copy(b_hbm.at[si_v.at[j + 3]], b1, semIB1)
            return 0

        lax.fori_loop(0, (nb - j0) // 2, body, 0)
        pltpu.make_async_copy(a0, outa_hbm.at[pl.ds(base, bs)], semOA0).wait()
        pltpu.make_async_copy(a1, outa_hbm.at[pl.ds(base, bs)], semOA1).wait()
        pltpu.make_async_copy(b0, outb_hbm.at[pl.ds(base, bs)], semOB0).wait()
        pltpu.make_async_copy(b1, outb_hbm.at[pl.ds(base, bs)], semOB1).wait()

    return gath(a, b, dest_g, src_g)


def _zero_rows(rbase):
    for zr in range(min(32, rbase), 7, -8):
        if rbase % zr == 0:
            return zr
    return 8


def _seg_sum_call(msg4, dest_r, n):
    np4, e, fc = msg4.shape
    nb, bs = dest_r.shape[1], dest_r.shape[2]
    epw = e // _NS
    hp = np4 // _NC
    rbase = (n // (8 * _NS)) * 8
    rem = n - rbase * _NS
    zr = _zero_rows(rbase)
    nzc = rbase // zr

    mesh = plsc.VectorSubcoreMesh(core_axis_name="c", subcore_axis_name="s")

    @functools.partial(
        pl.kernel,
        out_type=jax.ShapeDtypeStruct((np4, n, fc), jnp.float32),
        mesh=mesh,
        scratch_types=[
            pltpu.VMEM((nb, bs), jnp.int32),
            pltpu.VMEM((bs, fc), jnp.float32),
            pltpu.VMEM((bs, fc), jnp.float32),
            pltpu.VMEM((zr, fc), jnp.float32),
            pltpu.VMEM_SHARED((n, fc), jnp.float32),
            pltpu.SemaphoreType.DMA,
            pltpu.SemaphoreType.DMA,
        ],
    )
    def seg(msg_hbm, idx_hbm, out_hbm, idx_v, buf0, buf1, zbuf, acc, sem0,
            sem1):
        c = lax.axis_index("c")
        s = lax.axis_index("s")
        pltpu.sync_copy(idx_hbm.at[s], idx_v)

        def zrow(i, _):
            for l in range(fc // 16):
                zbuf[i, pl.ds(l * 16, 16)] = jnp.zeros((16,), jnp.float32)
            return 0
        lax.fori_loop(0, zr, zrow, 0)

        for half in range(hp):
            cp = c * hp + half

            for k in range(nzc):
                pltpu.sync_copy(zbuf, acc.at[pl.ds(s * rbase + k * zr, zr)])
            if rem:
                @pl.when(s == 0)
                def _():
                    pltpu.sync_copy(zbuf.at[pl.ds(0, rem)],
                                    acc.at[pl.ds(rbase * _NS, rem)])
            plsc.subcore_barrier()

            j0 = nb % 2
            if j0:
                pltpu.sync_copy(msg_hbm.at[cp, pl.ds(s * epw, bs)], buf0)
                pltpu.sync_copy(buf0, acc.at[idx_v.at[0]], add=True)
            pltpu.async_copy(msg_hbm.at[cp, pl.ds(s * epw + j0 * bs, bs)],
                             buf0, sem0)
            pltpu.async_copy(
                msg_hbm.at[cp, pl.ds(s * epw + (j0 + 1) * bs, bs)], buf1,
                sem1)

            def body(i, _):
                j = j0 + 2 * i
                pltpu.make_async_copy(msg_hbm.at[cp, pl.ds(0, bs)], buf0,
                                      sem0).wait()
                pltpu.sync_copy(buf0, acc.at[idx_v.at[j]], add=True)

                @pl.when(j + 2 < nb)
                def _():
                    pltpu.async_copy(
                        msg_hbm.at[cp, pl.ds(s * epw + (j + 2) * bs, bs)],
                        buf0, sem0)

                pltpu.make_async_copy(msg_hbm.at[cp, pl.ds(0, bs)], buf1,
                                      sem1).wait()
                pltpu.sync_copy(buf1, acc.at[idx_v.at[j + 1]], add=True)

                @pl.when(j + 3 < nb)
                def _():
                    pltpu.async_copy(
                        msg_hbm.at[cp, pl.ds(s * epw + (j + 3) * bs, bs)],
                        buf1, sem1)
                return 0

            lax.fori_loop(0, (nb - j0) // 2, body, 0)
            plsc.subcore_barrier()
            pltpu.sync_copy(acc.at[pl.ds(s * rbase, rbase)],
                            out_hbm.at[cp, pl.ds(s * rbase, rbase)])
            if rem:
                @pl.when(s == 0)
                def _():
                    pltpu.sync_copy(acc.at[pl.ds(rbase * _NS, rem)],
                                    out_hbm.at[cp, pl.ds(rbase * _NS, rem)])
            plsc.subcore_barrier()

    return seg(msg4, dest_r)


def _node_body(x_ref, *refs):
    nagg = len(refs) - 9
    agg_refs = refs[:nagg]
    (w1a_ref, w1b_ref, b1_ref, w2_ref, b2_ref, w3_ref, g_ref, be_ref,
     out_ref) = refs[nagg:]
    agg = jnp.concatenate(
        [agg_refs[0][i] for i in range(agg_refs[0].shape[0])], axis=-1)
    for ar in agg_refs[1:]:
        agg = agg + jnp.concatenate(
            [ar[i] for i in range(ar.shape[0])], axis=-1)
    h1 = _leaky(
        jnp.dot(x_ref[...], w1a_ref[...], preferred_element_type=jnp.float32)
        + jnp.dot(agg, w1b_ref[...],
                  preferred_element_type=jnp.float32)
        + b1_ref[...])
    h2 = _leaky(jnp.dot(h1, w2_ref[...], preferred_element_type=jnp.float32)
                + b2_ref[...])
    h3 = jnp.dot(h2, w3_ref[...], preferred_element_type=jnp.float32)
    out_ref[...] = _ln(h3, g_ref[...], be_ref[...])


def _node_call(x, agg4s, p):
    n, d_in = x.shape
    np4 = agg4s[0].shape[0]
    d_msg = np4 * _FC
    d_hid = p['W2'].shape[0]
    d_out = p['W3'].shape[1]
    tn = _row_tile(n)
    vec = lambda v: v.reshape(1, -1)
    return pl.pallas_call(
        _node_body,
        grid=(n // tn,),
        in_specs=[
            pl.BlockSpec((tn, d_in), lambda i: (i, 0)),
        ] + [
            pl.BlockSpec((np4, tn, _FC), lambda i: (0, i, 0))
            for _ in agg4s
        ] + [
            pl.BlockSpec((d_in, d_hid), lambda i: (0, 0)),
            pl.BlockSpec((d_msg, d_hid), lambda i: (0, 0)),
            pl.BlockSpec((1, d_hid), lambda i: (0, 0)),
            pl.BlockSpec((d_hid, d_hid), lambda i: (0, 0)),
            pl.BlockSpec((1, d_hid), lambda i: (0, 0)),
            pl.BlockSpec((d_hid, d_out), lambda i: (0, 0)),
            pl.BlockSpec((1, d_out), lambda i: (0, 0)),
            pl.BlockSpec((1, d_out), lambda i: (0, 0)),
        ],
        out_specs=pl.BlockSpec((tn, d_out), lambda i: (i, 0)),
        out_shape=jax.ShapeDtypeStruct((n, d_out), jnp.float32),
    )(x, *agg4s, p['W1'][:d_in], p['W1'][d_in:], vec(p['b1']), p['W2'],
      vec(p['b2']), p['W3'], vec(p['g']), vec(p['be']))


def _batch_size(epw, cap=128):
    for bs in range(cap, 7, -1):
        if bs % 8 == 0 and epw % bs == 0:
            return bs
    for bs in range(7, 0, -1):
        if epw % bs == 0:
            return bs
    return 1


def _chunk_sizes(e, n_chunks):
    unit = 40 * _NC * _NS
    units = e // unit
    if e % unit:
        unit = 8 * _NC * _NS
        units = e // unit
    if units < n_chunks or e % unit:
        return [e]
    base = units // n_chunks
    rem = units % n_chunks
    return [(base + (i < rem)) * unit for i in range(n_chunks)]


def kernel(x, edge_index, params):
    src = edge_index[0]
    dest = edge_index[1]
    n = x.shape[0]
    e = dest.shape[0]
    nw = _NC * _NS

    idx_chunks = []
    off = 0
    for ce in _chunk_sizes(e, 2):
        epg = ce // nw
        bsg = _batch_size(epg, cap=40)
        eps = ce // _NS
        bss = _batch_size(eps)
        dch = lax.slice_in_dim(dest, off, off + ce)
        sch = lax.slice_in_dim(src, off, off + ce)
        idx_chunks.append((
            dch.reshape(nw, epg // bsg, bsg),
            sch.reshape(nw, epg // bsg, bsg),
            dch.reshape(_NS, eps // bss, bss),
        ))
        off += ce

    for p in params:
        pe, pn = p['edge'], p['node']
        d_in = x.shape[1]
        wcat = jnp.concatenate([pe['W1'][:d_in], pe['W1'][d_in:]], axis=1)
        a, b = _ab_call(x, wcat)
        d_msg = a.shape[1]
        a32 = lax.bitcast_convert_type(
            a.reshape(n, d_msg // 2, 2), jnp.int32)
        b32 = lax.bitcast_convert_type(
            b.reshape(n, d_msg // 2, 2), jnp.int32)
        agg4s = []
        for dg, sg, dr in idx_chunks:
            ga32, gb32 = _gather_add_call(a32, b32, dg, sg)
            ga = lax.bitcast_convert_type(ga32, jnp.bfloat16).reshape(
                ga32.shape[0], d_msg)
            gb = lax.bitcast_convert_type(gb32, jnp.bfloat16).reshape(
                gb32.shape[0], d_msg)
            msg4 = _edge_tail_call(ga, gb, pe)
            agg4s.append(_seg_sum_call(msg4, dr, n))
        x = _node_call(x, agg4s, pn)
    return x


# speedup vs baseline: 5.5581x; 5.5581x over previous
import functools

import jax
import jax.numpy as jnp
from jax import lax
from jax.experimental import pallas as pl
from jax.experimental.pallas import tpu as pltpu
from jax.experimental.pallas import tpu_sc as plsc

EPS = 1e-5
_NC = 2
_NS = 16
_FC = 128


def _leaky(v):
    return jnp.where(v >= 0, v, 0.2 * v)


def _ln(h, g, b):
    mu = jnp.mean(h, axis=-1, keepdims=True)
    d = h - mu
    var = jnp.mean(d * d, axis=-1, keepdims=True)
    return d * lax.rsqrt(var + EPS) * g + b


def _pack_bf16(v):
    k = v.shape[1] // 2

    def rnd(x):
        u = lax.bitcast_convert_type(x, jnp.uint32)
        return (u + jnp.uint32(0x7FFF) + ((u >> 16) & jnp.uint32(1))) >> 16

    packed = rnd(v[:, :k]) | (rnd(v[:, k:]) << 16)
    return lax.bitcast_convert_type(packed, jnp.int32)


def _unpack_bf16(p):
    u = lax.bitcast_convert_type(p, jnp.uint32)
    lo = lax.bitcast_convert_type(u << 16, jnp.float32)
    hi = lax.bitcast_convert_type(u & jnp.uint32(0xFFFF0000), jnp.float32)
    return jnp.concatenate([lo, hi], axis=1)


def _row_tile(n, cap=1024):
    for t in range(cap, 7, -8):
        if n % t == 0:
            return t
    return n


def _ab_body(x_ref, w_ref, a_ref, b_ref):
    h = jnp.dot(x_ref[...], w_ref[...], preferred_element_type=jnp.float32)
    d = h.shape[1] // 2
    a_ref[...] = _pack_bf16(h[:, :d])
    b_ref[...] = _pack_bf16(h[:, d:])


def _ab_call(x, wcat):
    n, d_in = x.shape
    d2 = wcat.shape[1]
    d = d2 // 2
    tn = _row_tile(n)
    return pl.pallas_call(
        _ab_body,
        grid=(n // tn,),
        in_specs=[
            pl.BlockSpec((tn, d_in), lambda i: (i, 0)),
            pl.BlockSpec((d_in, d2), lambda i: (0, 0)),
        ],
        out_specs=[
            pl.BlockSpec((tn, d // 2), lambda i: (i, 0)),
            pl.BlockSpec((tn, d // 2), lambda i: (i, 0)),
        ],
        out_shape=[
            jax.ShapeDtypeStruct((n, d // 2), jnp.int32),
            jax.ShapeDtypeStruct((n, d // 2), jnp.int32),
        ],
    )(x, wcat)


def _edge_tail_body(ga_ref, gb_ref, b1_ref, w2_ref, b2_ref, w3_ref, g_ref,
                    be_ref, out_ref):
    h1 = _leaky(_unpack_bf16(ga_ref[...]) + _unpack_bf16(gb_ref[...])
                + b1_ref[...])
    h2 = _leaky(jnp.dot(h1, w2_ref[...], preferred_element_type=jnp.float32)
                + b2_ref[...])
    h3 = jnp.dot(h2, w3_ref[...], preferred_element_type=jnp.float32)
    msg = _ln(h3, g_ref[...], be_ref[...])
    for c4 in range(out_ref.shape[0]):
        out_ref[c4] = msg[:, c4 * _FC:(c4 + 1) * _FC]


def _edge_tail_call(ga, gb, p):
    e, dh = ga.shape
    d = dh * 2
    np4 = d // _FC
    te = _row_tile(e)
    vec = lambda v: v.reshape(1, -1)
    return pl.pallas_call(
        _edge_tail_body,
        grid=(e // te,),
        in_specs=[
            pl.BlockSpec((te, dh), lambda i: (i, 0)),
            pl.BlockSpec((te, dh), lambda i: (i, 0)),
            pl.BlockSpec((1, d), lambda i: (0, 0)),
            pl.BlockSpec((d, d), lambda i: (0, 0)),
            pl.BlockSpec((1, d), lambda i: (0, 0)),
            pl.BlockSpec((d, d), lambda i: (0, 0)),
            pl.BlockSpec((1, d), lambda i: (0, 0)),
            pl.BlockSpec((1, d), lambda i: (0, 0)),
        ],
        out_specs=pl.BlockSpec((np4, te, _FC), lambda i: (0, i, 0)),
        out_shape=jax.ShapeDtypeStruct((np4, e, _FC), jnp.float32),
    )(ga, gb, vec(p['b1']), p['W2'], vec(p['b2']), p['W3'], vec(p['g']),
      vec(p['be']))


def _gather_add_call(a, b, dest_g, src_g):
    n, d = a.shape
    nw = _NC * _NS
    _, nb, bs = dest_g.shape
    epw = nb * bs
    e = epw * nw

    mesh = plsc.VectorSubcoreMesh(core_axis_name="c", subcore_axis_name="s")

    @functools.partial(
        pl.kernel,
        out_type=[
            jax.ShapeDtypeStruct((e, d), jnp.int32),
            jax.ShapeDtypeStruct((e, d), jnp.int32),
        ],
        mesh=mesh,
        scratch_types=[
            pltpu.VMEM((nb, bs), jnp.int32),
            pltpu.VMEM((nb, bs), jnp.int32),
            pltpu.VMEM((bs, d), jnp.int32),
            pltpu.VMEM((bs, d), jnp.int32),
            pltpu.VMEM((bs, d), jnp.int32),
            pltpu.VMEM((bs, d), jnp.int32),
        ] + [pltpu.SemaphoreType.DMA] * 8,
    )
    def gath(a_hbm, b_hbm, di_hbm, si_hbm, outa_hbm, outb_hbm, di_v, si_v,
             a0, a1, b0, b1, semIA0, semIA1, semIB0, semIB1, semOA0, semOA1,
             semOB0, semOB1):
        c = lax.axis_index("c")
        s = lax.axis_index("s")
        wid = s * _NC + c
        base = wid * epw
        pltpu.sync_copy(di_hbm.at[wid], di_v)
        pltpu.sync_copy(si_hbm.at[wid], si_v)

        j0 = nb % 2
        if j0:
            pltpu.sync_copy(a_hbm.at[di_v.at[0]], a0)
            pltpu.sync_copy(a0, outa_hbm.at[pl.ds(base, bs)])
            pltpu.sync_copy(b_hbm.at[si_v.at[0]], b0)
            pltpu.sync_copy(b0, outb_hbm.at[pl.ds(base, bs)])

        pltpu.async_copy(a_hbm.at[di_v.at[j0]], a0, semIA0)
        pltpu.async_copy(b_hbm.at[si_v.at[j0]], b0, semIB0)
        pltpu.async_copy(a_hbm.at[di_v.at[j0 + 1]], a1, semIA1)
        pltpu.async_copy(b_hbm.at[si_v.at[j0 + 1]], b1, semIB1)

        def body(i, _):
            j = j0 + 2 * i
            pltpu.make_async_copy(a_hbm.at[di_v.at[0]], a0, semIA0).wait()
            pltpu.async_copy(a0, outa_hbm.at[pl.ds(base + j * bs, bs)],
                             semOA0)
            pltpu.make_async_copy(b_hbm.at[si_v.at[0]], b0, semIB0).wait()
            pltpu.async_copy(b0, outb_hbm.at[pl.ds(base + j * bs, bs)],
                             semOB0)
            pltpu.make_async_copy(a_hbm.at[di_v.at[0]], a1, semIA1).wait()
            pltpu.async_copy(a1, outa_hbm.at[pl.ds(base + (j + 1) * bs, bs)],
                             semOA1)
            pltpu.make_async_copy(b_hbm.at[si_v.at[0]], b1, semIB1).wait()
            pltpu.async_copy(b1, outb_hbm.at[pl.ds(base + (j + 1) * bs, bs)],
                             semOB1)

            @pl.when(j + 2 < nb)
            def _():
                pltpu.make_async_copy(a0, outa_hbm.at[pl.ds(base, bs)],
                                      semOA0).wait()
                pltpu.async_copy(a_hbm.at[di_v.at[j + 2]], a0, semIA0)
                pltpu.make_async_copy(b0, outb_hbm.at[pl.ds(base, bs)],
                                      semOB0).wait()
                pltpu.async_copy(b_hbm.at[si_v.at[j + 2]], b0, semIB0)

            @pl.when(j + 3 < nb)
            def _():
                pltpu.make_async_copy(a1, outa_hbm.at[pl.ds(base, bs)],
                                      semOA1).wait()
                pltpu.async_copy(a_hbm.at[di_v.at[j + 3]], a1, semIA1)
                pltpu.make_async_copy(b1, outb_hbm.at[pl.ds(base, bs)],
                                      semOB1).wait()
                pltpu.async_copy(b_hbm.at[si_v.at[j + 3]], b1, semIB1)
            return 0

        lax.fori_loop(0, (nb - j0) // 2, body, 0)
        pltpu.make_async_copy(a0, outa_hbm.at[pl.ds(base, bs)], semOA0).wait()
        pltpu.make_async_copy(a1, outa_hbm.at[pl.ds(base, bs)], semOA1).wait()
        pltpu.make_async_copy(b0, outb_hbm.at[pl.ds(base, bs)], semOB0).wait()
        pltpu.make_async_copy(b1, outb_hbm.at[pl.ds(base, bs)], semOB1).wait()

    return gath(a, b, dest_g, src_g)


def _zero_rows(rbase):
    for zr in range(min(32, rbase), 7, -8):
        if rbase % zr == 0:
            return zr
    return 8


def _seg_sum_call(msg4, dest_r, n):
    np4, e, fc = msg4.shape
    nb, bs = dest_r.shape[1], dest_r.shape[2]
    epw = e // _NS
    hp = np4 // _NC
    rbase = (n // (8 * _NS)) * 8
    rem = n - rbase * _NS
    zr = _zero_rows(rbase)
    nzc = rbase // zr

    mesh = plsc.VectorSubcoreMesh(core_axis_name="c", subcore_axis_name="s")

    @functools.partial(
        pl.kernel,
        out_type=jax.ShapeDtypeStruct((np4, n, fc), jnp.float32),
        mesh=mesh,
        scratch_types=[
            pltpu.VMEM((nb, bs), jnp.int32),
            pltpu.VMEM((bs, fc), jnp.float32),
            pltpu.VMEM((bs, fc), jnp.float32),
            pltpu.VMEM((zr, fc), jnp.float32),
            pltpu.VMEM_SHARED((n, fc), jnp.float32),
            pltpu.SemaphoreType.DMA,
            pltpu.SemaphoreType.DMA,
        ],
    )
    def seg(msg_hbm, idx_hbm, out_hbm, idx_v, buf0, buf1, zbuf, acc, sem0,
            sem1):
        c = lax.axis_index("c")
        s = lax.axis_index("s")
        pltpu.sync_copy(idx_hbm.at[s], idx_v)

        def zrow(i, _):
            for l in range(fc // 16):
                zbuf[i, pl.ds(l * 16, 16)] = jnp.zeros((16,), jnp.float32)
            return 0
        lax.fori_loop(0, zr, zrow, 0)

        for half in range(hp):
            cp = c * hp + half

            for k in range(nzc):
                pltpu.sync_copy(zbuf, acc.at[pl.ds(s * rbase + k * zr, zr)])
            if rem:
                @pl.when(s == 0)
                def _():
                    pltpu.sync_copy(zbuf.at[pl.ds(0, rem)],
                                    acc.at[pl.ds(rbase * _NS, rem)])
            plsc.subcore_barrier()

            j0 = nb % 2
            if j0:
                pltpu.sync_copy(msg_hbm.at[cp, pl.ds(s * epw, bs)], buf0)
                pltpu.sync_copy(buf0, acc.at[idx_v.at[0]], add=True)
            pltpu.async_copy(msg_hbm.at[cp, pl.ds(s * epw + j0 * bs, bs)],
                             buf0, sem0)
            pltpu.async_copy(
                msg_hbm.at[cp, pl.ds(s * epw + (j0 + 1) * bs, bs)], buf1,
                sem1)

            def body(i, _):
                j = j0 + 2 * i
                pltpu.make_async_copy(msg_hbm.at[cp, pl.ds(0, bs)], buf0,
                                      sem0).wait()
                pltpu.sync_copy(buf0, acc.at[idx_v.at[j]], add=True)

                @pl.when(j + 2 < nb)
                def _():
                    pltpu.async_copy(
                        msg_hbm.at[cp, pl.ds(s * epw + (j + 2) * bs, bs)],
                        buf0, sem0)

                pltpu.make_async_copy(msg_hbm.at[cp, pl.ds(0, bs)], buf1,
                                      sem1).wait()
                pltpu.sync_copy(buf1, acc.at[idx_v.at[j + 1]], add=True)

                @pl.when(j + 3 < nb)
                def _():
                    pltpu.async_copy(
                        msg_hbm.at[cp, pl.ds(s * epw + (j + 3) * bs, bs)],
                        buf1, sem1)
                return 0

            lax.fori_loop(0, (nb - j0) // 2, body, 0)
            plsc.subcore_barrier()
            pltpu.sync_copy(acc.at[pl.ds(s * rbase, rbase)],
                            out_hbm.at[cp, pl.ds(s * rbase, rbase)])
            if rem:
                @pl.when(s == 0)
                def _():
                    pltpu.sync_copy(acc.at[pl.ds(rbase * _NS, rem)],
                                    out_hbm.at[cp, pl.ds(rbase * _NS, rem)])
            plsc.subcore_barrier()

    return seg(msg4, dest_r)


def _node_body(x_ref, *refs):
    nagg = len(refs) - 9
    agg_refs = refs[:nagg]
    (w1a_ref, w1b_ref, b1_ref, w2_ref, b2_ref, w3_ref, g_ref, be_ref,
     out_ref) = refs[nagg:]
    agg = jnp.concatenate(
        [agg_refs[0][i] for i in range(agg_refs[0].shape[0])], axis=-1)
    for ar in agg_refs[1:]:
        agg = agg + jnp.concatenate(
            [ar[i] for i in range(ar.shape[0])], axis=-1)
    h1 = _leaky(
        jnp.dot(x_ref[...], w1a_ref[...], preferred_element_type=jnp.float32)
        + jnp.dot(agg, w1b_ref[...],
                  preferred_element_type=jnp.float32)
        + b1_ref[...])
    h2 = _leaky(jnp.dot(h1, w2_ref[...], preferred_element_type=jnp.float32)
                + b2_ref[...])
    h3 = jnp.dot(h2, w3_ref[...], preferred_element_type=jnp.float32)
    out_ref[...] = _ln(h3, g_ref[...], be_ref[...])


def _node_call(x, agg4s, p):
    n, d_in = x.shape
    np4 = agg4s[0].shape[0]
    d_msg = np4 * _FC
    d_hid = p['W2'].shape[0]
    d_out = p['W3'].shape[1]
    tn = _row_tile(n)
    vec = lambda v: v.reshape(1, -1)
    return pl.pallas_call(
        _node_body,
        grid=(n // tn,),
        in_specs=[
            pl.BlockSpec((tn, d_in), lambda i: (i, 0)),
        ] + [
            pl.BlockSpec((np4, tn, _FC), lambda i: (0, i, 0))
            for _ in agg4s
        ] + [
            pl.BlockSpec((d_in, d_hid), lambda i: (0, 0)),
            pl.BlockSpec((d_msg, d_hid), lambda i: (0, 0)),
            pl.BlockSpec((1, d_hid), lambda i: (0, 0)),
            pl.BlockSpec((d_hid, d_hid), lambda i: (0, 0)),
            pl.BlockSpec((1, d_hid), lambda i: (0, 0)),
            pl.BlockSpec((d_hid, d_out), lambda i: (0, 0)),
            pl.BlockSpec((1, d_out), lambda i: (0, 0)),
            pl.BlockSpec((1, d_out), lambda i: (0, 0)),
        ],
        out_specs=pl.BlockSpec((tn, d_out), lambda i: (i, 0)),
        out_shape=jax.ShapeDtypeStruct((n, d_out), jnp.float32),
    )(x, *agg4s, p['W1'][:d_in], p['W1'][d_in:], vec(p['b1']), p['W2'],
      vec(p['b2']), p['W3'], vec(p['g']), vec(p['be']))


def _batch_size(epw, cap=128):
    for bs in range(cap, 7, -1):
        if bs % 8 == 0 and epw % bs == 0:
            return bs
    for bs in range(7, 0, -1):
        if epw % bs == 0:
            return bs
    return 1


def _chunk_sizes(e, n_chunks):
    unit = 40 * _NC * _NS
    units = e // unit
    if e % unit:
        unit = 8 * _NC * _NS
        units = e // unit
    if units < n_chunks or e % unit:
        return [e]
    base = units // n_chunks
    rem = units % n_chunks
    return [(base + (i < rem)) * unit for i in range(n_chunks)]


def kernel(x, edge_index, params):
    src = edge_index[0]
    dest = edge_index[1]
    n = x.shape[0]
    e = dest.shape[0]
    nw = _NC * _NS

    idx_chunks = []
    off = 0
    for ce in _chunk_sizes(e, 2):
        epg = ce // nw
        bsg = _batch_size(epg, cap=40)
        eps = ce // _NS
        bss = _batch_size(eps)
        dch = lax.slice_in_dim(dest, off, off + ce)
        sch = lax.slice_in_dim(src, off, off + ce)
        idx_chunks.append((
            dch.reshape(nw, epg // bsg, bsg),
            sch.reshape(nw, epg // bsg, bsg),
            dch.reshape(_NS, eps // bss, bss),
        ))
        off += ce

    for p in params:
        pe, pn = p['edge'], p['node']
        d_in = x.shape[1]
        wcat = jnp.concatenate([pe['W1'][:d_in], pe['W1'][d_in:]], axis=1)
        a32, b32 = _ab_call(x, wcat)
        agg4s = []
        for dg, sg, dr in idx_chunks:
            ga32, gb32 = _gather_add_call(a32, b32, dg, sg)
            msg4 = _edge_tail_call(ga32, gb32, pe)
            agg4s.append(_seg_sum_call(msg4, dr, n))
        x = _node_call(x, agg4s, pn)
    return x
